# baseline (speedup 1.0000x reference)
import jax
import jax.numpy as jnp
from jax import lax
from jax.experimental import pallas as pl
from jax.experimental.pallas import tpu as pltpu
from jax.experimental.pallas import tpu_sc as plsc

_N = 10000
_E = 320000
_D = 128
_HD = _D // 2
_NEG = 0.2
_NT = 16
_EPT = _E // _NT
_CH = 80
_NCH = _EPT // _CH
_SL = 50
_NSL = _NCH // _SL
_RPT = _N // _NT
_ZR = 125
_BN = 5000
_GB = _N // _BN


def _eler_block(el, er):
    row = lax.broadcasted_iota(jnp.int32, (8, _BN), 0)
    return jnp.where(row == 0, el[None, :],
                     jnp.where(row == 1, er[None, :], 0.0))[None]


def _tc_mm_body(x_ref, wl_ref, wh_ref, al_ref, ar_ref, h_ref, eler_ref):
    x = x_ref[...]
    hl = jnp.dot(x, wl_ref[...], preferred_element_type=jnp.float32)
    hh = jnp.dot(x, wh_ref[...], preferred_element_type=jnp.float32)
    h_ref[0, :, :] = hl
    h_ref[1, :, :] = hh
    h = jnp.concatenate([hl, hh], axis=1)
    el = jnp.sum(h * al_ref[...], axis=1)
    er = jnp.sum(h * ar_ref[...], axis=1)
    eler_ref[...] = _eler_block(el, er)


def _tc_mm(x, w, al, ar):
    return pl.pallas_call(
        _tc_mm_body,
        grid=(_GB,),
        in_specs=[
            pl.BlockSpec((_BN, _D), lambda i: (i, 0)),
            pl.BlockSpec((_D, _HD), lambda i: (0, 0)),
            pl.BlockSpec((_D, _HD), lambda i: (0, 0)),
            pl.BlockSpec((1, _D), lambda i: (0, 0)),
            pl.BlockSpec((1, _D), lambda i: (0, 0)),
        ],
        out_specs=[
            pl.BlockSpec((2, _BN, _HD), lambda i: (0, i, 0)),
            pl.BlockSpec((1, 8, _BN), lambda i: (i, 0, 0)),
        ],
        out_shape=[
            jax.ShapeDtypeStruct((2, _N, _HD), jnp.float32),
            jax.ShapeDtypeStruct((_GB, 8, _BN), jnp.float32),
        ],
    )(x, w[:, :_HD], w[:, _HD:], al, ar)


def _tc_mm2_body(hl_ref, hh_ref, wll_ref, wlh_ref, whl_ref, whh_ref,
                 al_ref, ar_ref, h_ref, eler_ref):
    a = hl_ref[0]
    b = hh_ref[0]
    hl = (jnp.dot(a, wll_ref[...], preferred_element_type=jnp.float32)
          + jnp.dot(b, whl_ref[...], preferred_element_type=jnp.float32))
    hh = (jnp.dot(a, wlh_ref[...], preferred_element_type=jnp.float32)
          + jnp.dot(b, whh_ref[...], preferred_element_type=jnp.float32))
    h_ref[0, :, :] = hl
    h_ref[1, :, :] = hh
    h = jnp.concatenate([hl, hh], axis=1)
    el = jnp.sum(h * al_ref[...], axis=1)
    er = jnp.sum(h * ar_ref[...], axis=1)
    eler_ref[...] = _eler_block(el, er)


def _tc_mm2(hpair, w, al, ar):
    return pl.pallas_call(
        _tc_mm2_body,
        grid=(_GB,),
        in_specs=[
            pl.BlockSpec((1, _BN, _HD), lambda i: (0, i, 0)),
            pl.BlockSpec((1, _BN, _HD), lambda i: (1, i, 0)),
            pl.BlockSpec((_HD, _HD), lambda i: (0, 0)),
            pl.BlockSpec((_HD, _HD), lambda i: (0, 0)),
            pl.BlockSpec((_HD, _HD), lambda i: (0, 0)),
            pl.BlockSpec((_HD, _HD), lambda i: (0, 0)),
            pl.BlockSpec((1, _D), lambda i: (0, 0)),
            pl.BlockSpec((1, _D), lambda i: (0, 0)),
        ],
        out_specs=[
            pl.BlockSpec((2, _BN, _HD), lambda i: (0, i, 0)),
            pl.BlockSpec((1, 8, _BN), lambda i: (i, 0, 0)),
        ],
        out_shape=[
            jax.ShapeDtypeStruct((2, _N, _HD), jnp.float32),
            jax.ShapeDtypeStruct((_GB, 8, _BN), jnp.float32),
        ],
    )(hpair, hpair, w[:_HD, :_HD], w[:_HD, _HD:], w[_HD:, :_HD],
      w[_HD:, _HD:], al, ar)


def _sc_msg_body(h_hbm, eler_hbm, src_hbm, dst_hbm, b_hbm, out_hbm,
                 src_v, dst_v, el_v, er_v, b_v, gbuf, ebuf, sbuf, zbuf, zbuf2,
                 accm, accs, gsem0, gsem1, msem, psem):
    c = lax.axis_index("c")
    s = lax.axis_index("s")

    zero16 = jnp.zeros((16,), jnp.float32)

    def _zrow(r, carry):
        for j in range(_HD // 16):
            zbuf[r, pl.ds(j * 16, 16)] = zero16
        zbuf2[r, pl.ds(0, 16)] = zero16
        return carry
    lax.fori_loop(0, _ZR, _zrow, 0)

    def _zsb(r, carry):
        sbuf[0, r, pl.ds(0, 16)] = zero16
        sbuf[1, r, pl.ds(0, 16)] = zero16
        return carry
    lax.fori_loop(0, _CH, _zsb, 0)
    for k in range(_RPT // _ZR):
        pltpu.sync_copy(zbuf, accm.at[pl.ds(s * _RPT + k * _ZR, _ZR)])
        pltpu.sync_copy(zbuf2, accs.at[pl.ds(s * _RPT + k * _ZR, _ZR)])
    plsc.subcore_barrier()

    for g in range(_GB):
        pltpu.sync_copy(eler_hbm.at[g, 0], el_v.at[pl.ds(g * _BN, _BN)])
        pltpu.sync_copy(eler_hbm.at[g, 1], er_v.at[pl.ds(g * _BN, _BN)])
    pltpu.sync_copy(b_hbm.at[c], b_v)

    zeros16i = jnp.zeros((16,), jnp.int32)
    iota16 = lax.iota(jnp.int32, 16)
    gsems = (gsem0, gsem1)

    def _slab(sl, carry):
        pltpu.sync_copy(src_hbm.at[s, pl.ds(sl * _SL, _SL)], src_v)
        pltpu.sync_copy(dst_hbm.at[s, pl.ds(sl * _SL, _SL)], dst_v)
        pltpu.async_copy(h_hbm.at[c].at[src_v.at[0]], gbuf.at[0], gsem0)

        def _pair(cj, carry1):
            for b in range(2):
                ci = 2 * cj + b
                nb = 1 - b

                @pl.when(ci + 1 < _SL)
                def _prefetch():
                    pltpu.async_copy(h_hbm.at[c].at[src_v.at[ci + 1]],
                                     gbuf.at[nb], gsems[nb])

                @pl.when(sl * _SL + ci >= 2)
                def _drainp():
                    pltpu.make_async_copy(sbuf.at[b],
                                          accs.at[pl.ds(0, _CH)], psem).wait()

                for k in range(_CH // 16):
                    sv = src_v[ci, pl.ds(k * 16, 16)]
                    dv = dst_v[ci, pl.ds(k * 16, 16)]
                    e = (plsc.load_gather(el_v, [sv])
                         + plsc.load_gather(er_v, [dv]))
                    e = jnp.where(e > 0.0, e, _NEG * e)
                    plsc.store_scatter(
                        sbuf, [zeros16i + b, iota16 + k * 16, zeros16i],
                        jnp.exp(e))

                pltpu.make_async_copy(h_hbm.at[c].at[src_v.at[ci]],
                                      gbuf.at[b], gsems[b]).wait()

                @pl.when(sl * _SL + ci >= 2)
                def _drainm():
                    pltpu.make_async_copy(ebuf.at[b],
                                          accm.at[pl.ds(0, _CH)], msem).wait()

                @plsc.parallel_loop(0, _CH, unroll=8)
                def _edge(ei):
                    pi = plsc.load_gather(
                        sbuf, [zeros16i + b, zeros16i + ei, zeros16i])
                    for j in range(_HD // 16):
                        ebuf[b, ei, pl.ds(j * 16, 16)] = (
                            gbuf[b, ei, pl.ds(j * 16, 16)] * pi)

                pltpu.async_copy(ebuf.at[b], accm.at[dst_v.at[ci]], msem,
                                 add=True)
                pltpu.async_copy(sbuf.at[b], accs.at[dst_v.at[ci]], psem,
                                 add=True)
            return carry1
        lax.fori_loop(0, _SL // 2, _pair, 0)
        return carry
    lax.fori_loop(0, _NSL, _slab, 0)

    for _ in range(2):
        pltpu.make_async_copy(ebuf.at[0], accm.at[pl.ds(0, _CH)], msem).wait()
        pltpu.make_async_copy(sbuf.at[0], accs.at[pl.ds(0, _CH)], psem).wait()

    plsc.subcore_barrier()
    bvecs = [b_v[pl.ds(j * 16, 16)] for j in range(_HD // 16)]
    for k in range(_RPT // _ZR):
        r0 = s * _RPT + k * _ZR
        pltpu.sync_copy(accm.at[pl.ds(r0, _ZR)], zbuf)
        pltpu.sync_copy(accs.at[pl.ds(r0, _ZR)], zbuf2)

        def _norm(r, carry):
            sv = plsc.load_gather(zbuf2, [zeros16i + r, zeros16i])
            inv = 1.0 / (sv + 1e-9)
            for j in range(_HD // 16):
                zbuf[r, pl.ds(j * 16, 16)] = jnp.maximum(
                    zbuf[r, pl.ds(j * 16, 16)] * inv + bvecs[j], 0.0)
            return carry
        lax.fori_loop(0, _ZR, _norm, 0)
        pltpu.sync_copy(zbuf, out_hbm.at[c, pl.ds(r0, _ZR)])


_sc_msg = pl.kernel(
    _sc_msg_body,
    out_type=jax.ShapeDtypeStruct((2, _N, _HD), jnp.float32),
    mesh=plsc.VectorSubcoreMesh(core_axis_name="c", subcore_axis_name="s",
                                num_cores=2, num_subcores=16),
    compiler_params=pltpu.CompilerParams(use_tc_tiling_on_sc=False,
                                         needs_layout_passes=False),
    scratch_types=[
        pltpu.VMEM((_SL, _CH), jnp.int32),
        pltpu.VMEM((_SL, _CH), jnp.int32),
        pltpu.VMEM((_N,), jnp.float32),
        pltpu.VMEM((_N,), jnp.float32),
        pltpu.VMEM((_HD,), jnp.float32),
        pltpu.VMEM((2, _CH, _HD), jnp.float32),
        pltpu.VMEM((2, _CH, _HD), jnp.float32),
        pltpu.VMEM((2, _CH, 16), jnp.float32),
        pltpu.VMEM((_ZR, _HD), jnp.float32),
        pltpu.VMEM((_ZR, 16), jnp.float32),
        pltpu.VMEM_SHARED((_N, _HD), jnp.float32),
        pltpu.VMEM_SHARED((_N, 16), jnp.float32),
        pltpu.SemaphoreType.DMA,
        pltpu.SemaphoreType.DMA,
        pltpu.SemaphoreType.DMA,
        pltpu.SemaphoreType.DMA,
    ],
)


def kernel(feat, edge_index, W1, attn_l1, attn_r1, b1, W2, attn_l2, attn_r2, b2):
    src = edge_index[0].reshape(_NT, _NCH, _CH)
    dst = edge_index[1].reshape(_NT, _NCH, _CH)

    h, eler = _tc_mm(feat, W1, attn_l1, attn_r1)
    H1 = _sc_msg(h, eler, src, dst, b1.reshape(2, _HD))
    hmid, eler2 = _tc_mm2(H1, W2, attn_l2, attn_r2)
    H2 = _sc_msg(hmid, eler2, src, dst, b2.reshape(2, _HD))
    h1 = jnp.concatenate([H1[0], H1[1]], axis=1)
    h2 = jnp.concatenate([H2[0], H2[1]], axis=1)
    return (h1, h2)


# speedup vs baseline: 1.0152x; 1.0152x over previous
import jax
import jax.numpy as jnp
from jax import lax
from jax.experimental import pallas as pl
from jax.experimental.pallas import tpu as pltpu
from jax.experimental.pallas import tpu_sc as plsc

_N = 10000
_E = 320000
_D = 128
_HD = _D // 2
_NEG = 0.2
_NT = 16
_EPT = _E // _NT
_CH = 80
_NCH = _EPT // _CH
_SL = 50
_NSL = _NCH // _SL
_RPT = _N // _NT
_ZR = 125
_BN = 2000
_GB = _N // _BN


def _eler_block(el, er):
    row = lax.broadcasted_iota(jnp.int32, (8, _BN), 0)
    return jnp.where(row == 0, el[None, :],
                     jnp.where(row == 1, er[None, :], 0.0))[None]


def _tc_mm_body(x_ref, wl_ref, wh_ref, al_ref, ar_ref, h_ref, eler_ref):
    x = x_ref[...]
    hl = jnp.dot(x, wl_ref[...], preferred_element_type=jnp.float32)
    hh = jnp.dot(x, wh_ref[...], preferred_element_type=jnp.float32)
    h_ref[0, :, :] = hl
    h_ref[1, :, :] = hh
    h = jnp.concatenate([hl, hh], axis=1)
    el = jnp.sum(h * al_ref[...], axis=1)
    er = jnp.sum(h * ar_ref[...], axis=1)
    eler_ref[...] = _eler_block(el, er)


def _tc_mm(x, w, al, ar):
    return pl.pallas_call(
        _tc_mm_body,
        grid=(_GB,),
        in_specs=[
            pl.BlockSpec((_BN, _D), lambda i: (i, 0)),
            pl.BlockSpec((_D, _HD), lambda i: (0, 0)),
            pl.BlockSpec((_D, _HD), lambda i: (0, 0)),
            pl.BlockSpec((1, _D), lambda i: (0, 0)),
            pl.BlockSpec((1, _D), lambda i: (0, 0)),
        ],
        out_specs=[
            pl.BlockSpec((2, _BN, _HD), lambda i: (0, i, 0)),
            pl.BlockSpec((1, 8, _BN), lambda i: (i, 0, 0)),
        ],
        out_shape=[
            jax.ShapeDtypeStruct((2, _N, _HD), jnp.float32),
            jax.ShapeDtypeStruct((_GB, 8, _BN), jnp.float32),
        ],
    )(x, w[:, :_HD], w[:, _HD:], al, ar)


def _tc_mm2_body(hl_ref, hh_ref, wll_ref, wlh_ref, whl_ref, whh_ref,
                 al_ref, ar_ref, h_ref, eler_ref):
    a = hl_ref[0]
    b = hh_ref[0]
    hl = (jnp.dot(a, wll_ref[...], preferred_element_type=jnp.float32)
          + jnp.dot(b, whl_ref[...], preferred_element_type=jnp.float32))
    hh = (jnp.dot(a, wlh_ref[...], preferred_element_type=jnp.float32)
          + jnp.dot(b, whh_ref[...], preferred_element_type=jnp.float32))
    h_ref[0, :, :] = hl
    h_ref[1, :, :] = hh
    h = jnp.concatenate([hl, hh], axis=1)
    el = jnp.sum(h * al_ref[...], axis=1)
    er = jnp.sum(h * ar_ref[...], axis=1)
    eler_ref[...] = _eler_block(el, er)


def _tc_mm2(hpair, w, al, ar):
    return pl.pallas_call(
        _tc_mm2_body,
        grid=(_GB,),
        in_specs=[
            pl.BlockSpec((1, _BN, _HD), lambda i: (0, i, 0)),
            pl.BlockSpec((1, _BN, _HD), lambda i: (1, i, 0)),
            pl.BlockSpec((_HD, _HD), lambda i: (0, 0)),
            pl.BlockSpec((_HD, _HD), lambda i: (0, 0)),
            pl.BlockSpec((_HD, _HD), lambda i: (0, 0)),
            pl.BlockSpec((_HD, _HD), lambda i: (0, 0)),
            pl.BlockSpec((1, _D), lambda i: (0, 0)),
            pl.BlockSpec((1, _D), lambda i: (0, 0)),
        ],
        out_specs=[
            pl.BlockSpec((2, _BN, _HD), lambda i: (0, i, 0)),
            pl.BlockSpec((1, 8, _BN), lambda i: (i, 0, 0)),
        ],
        out_shape=[
            jax.ShapeDtypeStruct((2, _N, _HD), jnp.float32),
            jax.ShapeDtypeStruct((_GB, 8, _BN), jnp.float32),
        ],
    )(hpair, hpair, w[:_HD, :_HD], w[:_HD, _HD:], w[_HD:, :_HD],
      w[_HD:, _HD:], al, ar)


def _sc_msg_body(h_hbm, eler_hbm, src_hbm, dst_hbm, b_hbm, out_hbm,
                 src_v, dst_v, el_v, er_v, b_v, gbuf, ebuf, sbuf, zbuf, zbuf2,
                 accm, accs, gsem0, gsem1, msem, psem):
    c = lax.axis_index("c")
    s = lax.axis_index("s")

    zero16 = jnp.zeros((16,), jnp.float32)

    def _zrow(r, carry):
        for j in range(_HD // 16):
            zbuf[r, pl.ds(j * 16, 16)] = zero16
        zbuf2[r, pl.ds(0, 16)] = zero16
        return carry
    lax.fori_loop(0, _ZR, _zrow, 0)

    def _zsb(r, carry):
        sbuf[0, r, pl.ds(0, 16)] = zero16
        sbuf[1, r, pl.ds(0, 16)] = zero16
        return carry
    lax.fori_loop(0, _CH, _zsb, 0)
    for k in range(_RPT // _ZR):
        pltpu.sync_copy(zbuf, accm.at[pl.ds(s * _RPT + k * _ZR, _ZR)])
        pltpu.sync_copy(zbuf2, accs.at[pl.ds(s * _RPT + k * _ZR, _ZR)])
    plsc.subcore_barrier()

    for g in range(_GB):
        pltpu.sync_copy(eler_hbm.at[g, 0], el_v.at[pl.ds(g * _BN, _BN)])
        pltpu.sync_copy(eler_hbm.at[g, 1], er_v.at[pl.ds(g * _BN, _BN)])
    pltpu.sync_copy(b_hbm.at[c], b_v)

    zeros16i = jnp.zeros((16,), jnp.int32)
    iota16 = lax.iota(jnp.int32, 16)
    gsems = (gsem0, gsem1)

    def _slab(sl, carry):
        pltpu.sync_copy(src_hbm.at[s, pl.ds(sl * _SL, _SL)], src_v)
        pltpu.sync_copy(dst_hbm.at[s, pl.ds(sl * _SL, _SL)], dst_v)
        pltpu.async_copy(h_hbm.at[c].at[src_v.at[0]], gbuf.at[0], gsem0)

        def _pair(cj, carry1):
            for b in range(2):
                ci = 2 * cj + b
                nb = 1 - b

                @pl.when(ci + 1 < _SL)
                def _prefetch():
                    pltpu.async_copy(h_hbm.at[c].at[src_v.at[ci + 1]],
                                     gbuf.at[nb], gsems[nb])

                @pl.when(sl * _SL + ci >= 2)
                def _drainp():
                    pltpu.make_async_copy(sbuf.at[b],
                                          accs.at[pl.ds(0, _CH)], psem).wait()

                for k in range(_CH // 16):
                    sv = src_v[ci, pl.ds(k * 16, 16)]
                    dv = dst_v[ci, pl.ds(k * 16, 16)]
                    e = (plsc.load_gather(el_v, [sv])
                         + plsc.load_gather(er_v, [dv]))
                    e = jnp.where(e > 0.0, e, _NEG * e)
                    plsc.store_scatter(
                        sbuf, [zeros16i + b, iota16 + k * 16, zeros16i],
                        jnp.exp(e))

                pltpu.make_async_copy(h_hbm.at[c].at[src_v.at[ci]],
                                      gbuf.at[b], gsems[b]).wait()

                @pl.when(sl * _SL + ci >= 2)
                def _drainm():
                    pltpu.make_async_copy(ebuf.at[b],
                                          accm.at[pl.ds(0, _CH)], msem).wait()

                @plsc.parallel_loop(0, _CH, unroll=8)
                def _edge(ei):
                    pi = plsc.load_gather(
                        sbuf, [zeros16i + b, zeros16i + ei, zeros16i])
                    for j in range(_HD // 16):
                        ebuf[b, ei, pl.ds(j * 16, 16)] = (
                            gbuf[b, ei, pl.ds(j * 16, 16)] * pi)

                pltpu.async_copy(ebuf.at[b], accm.at[dst_v.at[ci]], msem,
                                 add=True)
                pltpu.async_copy(sbuf.at[b], accs.at[dst_v.at[ci]], psem,
                                 add=True)
            return carry1
        lax.fori_loop(0, _SL // 2, _pair, 0)
        return carry
    lax.fori_loop(0, _NSL, _slab, 0)

    for _ in range(2):
        pltpu.make_async_copy(ebuf.at[0], accm.at[pl.ds(0, _CH)], msem).wait()
        pltpu.make_async_copy(sbuf.at[0], accs.at[pl.ds(0, _CH)], psem).wait()

    plsc.subcore_barrier()
    bvecs = [b_v[pl.ds(j * 16, 16)] for j in range(_HD // 16)]
    for k in range(_RPT // _ZR):
        r0 = s * _RPT + k * _ZR
        pltpu.sync_copy(accm.at[pl.ds(r0, _ZR)], zbuf)
        pltpu.sync_copy(accs.at[pl.ds(r0, _ZR)], zbuf2)

        def _norm(r, carry):
            sv = plsc.load_gather(zbuf2, [zeros16i + r, zeros16i])
            inv = 1.0 / (sv + 1e-9)
            for j in range(_HD // 16):
                zbuf[r, pl.ds(j * 16, 16)] = jnp.maximum(
                    zbuf[r, pl.ds(j * 16, 16)] * inv + bvecs[j], 0.0)
            return carry
        lax.fori_loop(0, _ZR, _norm, 0)
        pltpu.sync_copy(zbuf, out_hbm.at[c, pl.ds(r0, _ZR)])


_sc_msg = pl.kernel(
    _sc_msg_body,
    out_type=jax.ShapeDtypeStruct((2, _N, _HD), jnp.float32),
    mesh=plsc.VectorSubcoreMesh(core_axis_name="c", subcore_axis_name="s",
                                num_cores=2, num_subcores=16),
    compiler_params=pltpu.CompilerParams(use_tc_tiling_on_sc=False,
                                         needs_layout_passes=False),
    scratch_types=[
        pltpu.VMEM((_SL, _CH), jnp.int32),
        pltpu.VMEM((_SL, _CH), jnp.int32),
        pltpu.VMEM((_N,), jnp.float32),
        pltpu.VMEM((_N,), jnp.float32),
        pltpu.VMEM((_HD,), jnp.float32),
        pltpu.VMEM((2, _CH, _HD), jnp.float32),
        pltpu.VMEM((2, _CH, _HD), jnp.float32),
        pltpu.VMEM((2, _CH, 16), jnp.float32),
        pltpu.VMEM((_ZR, _HD), jnp.float32),
        pltpu.VMEM((_ZR, 16), jnp.float32),
        pltpu.VMEM_SHARED((_N, _HD), jnp.float32),
        pltpu.VMEM_SHARED((_N, 16), jnp.float32),
        pltpu.SemaphoreType.DMA,
        pltpu.SemaphoreType.DMA,
        pltpu.SemaphoreType.DMA,
        pltpu.SemaphoreType.DMA,
    ],
)


def kernel(feat, edge_index, W1, attn_l1, attn_r1, b1, W2, attn_l2, attn_r2, b2):
    src = edge_index[0].reshape(_NT, _NCH, _CH)
    dst = edge_index[1].reshape(_NT, _NCH, _CH)

    h, eler = _tc_mm(feat, W1, attn_l1, attn_r1)
    H1 = _sc_msg(h, eler, src, dst, b1.reshape(2, _HD))
    hmid, eler2 = _tc_mm2(H1, W2, attn_l2, attn_r2)
    H2 = _sc_msg(hmid, eler2, src, dst, b2.reshape(2, _HD))
    h1 = jnp.concatenate([H1[0], H1[1]], axis=1)
    h2 = jnp.concatenate([H2[0], H2[1]], axis=1)
    return (h1, h2)
